# baseline (speedup 1.0000x reference)
import functools

import jax
import jax.numpy as jnp
from jax import lax
from jax.experimental import pallas as pl
from jax.experimental.pallas import tpu as pltpu
from jax.experimental.pallas import tpu_sc as plsc

E = 320000
R = 10000
RP = 10240
D = 128
NC = 2
NS = 16
NWORK = NC * NS
EPT = E // NWORK
W = 64
NB = 2
NFULL = EPT // W
REM = EPT - NFULL * W
NWIN = NFULL + (1 if REM else 0)
NWINP = -(-NWIN // NB) * NB
TAIL = EPT - W
RPT = RP // NS
LANES = 16
ZR = 16
NGRP = NWINP // NB


def _win_off(j):
    off = jnp.where(j < NFULL, j * W, TAIL if REM else 0)
    if REM:
        off = jnp.where(j > NFULL, 0, off)
    return off


def _sc_scatter_body(
    i1_hbm, idx_hbm, out_hbm, idx_v, upd_v, zrow_v, acc_sh, isem, gsems
):
    c = lax.axis_index("c")
    s = lax.axis_index("s")
    wid = c * NS + s
    ebase = wid * EPT

    idx_cp = pltpu.async_copy(idx_hbm.at[wid], idx_v, isem)
    prime = [
        pltpu.async_copy(
            i1_hbm.at[pl.ds(ebase + b * W, W)], upd_v.at[b], gsems[b]
        )
        for b in range(NB)
    ]

    def zero_row(i, carry):
        for blk in range(D // LANES):
            zrow_v[i, pl.ds(blk * LANES, LANES)] = jnp.zeros((LANES,), jnp.float32)
        return carry

    lax.fori_loop(0, ZR, zero_row, 0)
    for r in range(RPT // ZR):
        pltpu.sync_copy(zrow_v, acc_sh.at[pl.ds(s * RPT + r * ZR, ZR)])
    idx_cp.wait()
    plsc.subcore_barrier()

    def group(g, carry):
        for b in range(NB):
            j = g * NB + b
            prime[b].wait()
            pltpu.sync_copy(upd_v.at[b], acc_sh.at[idx_v.at[j]], add=True)
            off = _win_off(j + NB)
            pltpu.async_copy(
                i1_hbm.at[pl.ds(ebase + off, W)], upd_v.at[b], gsems[b]
            )
        return carry

    lax.fori_loop(0, NGRP - 1, group, 0)
    base = (NGRP - 1) * NB
    for b in range(NB):
        prime[b].wait()
        pltpu.sync_copy(upd_v.at[b], acc_sh.at[idx_v.at[base + b]], add=True)
    plsc.subcore_barrier()

    rbase = s * RPT
    pltpu.sync_copy(
        acc_sh.at[pl.ds(rbase, RPT)],
        out_hbm.at[c, pl.ds(rbase, RPT)],
    )


_sc_scatter = functools.partial(
    pl.kernel,
    out_type=jax.ShapeDtypeStruct((NC, RP, D), jnp.float32),
    mesh=plsc.VectorSubcoreMesh(
        core_axis_name="c", subcore_axis_name="s", num_cores=NC, num_subcores=NS
    ),
    scratch_types=[
        pltpu.VMEM((NWINP, W), jnp.int32),
        pltpu.VMEM((NB, W, D), jnp.float32),
        pltpu.VMEM((ZR, D), jnp.float32),
        pltpu.VMEM_SHARED((RP, D), jnp.float32),
        pltpu.SemaphoreType.DMA,
        [pltpu.SemaphoreType.DMA] * NB,
    ],
)(_sc_scatter_body)


def _sum_partials_body(a_ref, b_ref, o_ref):
    o_ref[...] = a_ref[0] + b_ref[0]


def kernel(i1, pair_i, p1):
    del p1
    pi = pair_i.astype(jnp.int32).reshape(NWORK, EPT)
    ntrash = (W - REM if REM else 0) + (NWINP - NWIN) * W
    parts = [pi[:, : NFULL * W].reshape(NWORK, NFULL, W)]
    if ntrash:
        trash = R + (
            jnp.arange(NWORK * ntrash, dtype=jnp.int32) % (RP - R)
        ).reshape(NWORK, ntrash)
        rest = jnp.concatenate([trash[:, : W - REM], pi[:, NFULL * W :]], axis=1) \
            if REM else trash[:, :0]
        rest = jnp.concatenate(
            [rest, trash[:, W - REM if REM else 0 :]], axis=1
        ).reshape(NWORK, NWINP - NFULL, W)
        parts.append(rest)
    idx = jnp.concatenate(parts, axis=1) if len(parts) > 1 else parts[0]
    partials = _sc_scatter(i1, idx)
    blk = 1000
    out = pl.pallas_call(
        _sum_partials_body,
        out_shape=jax.ShapeDtypeStruct((R, D), jnp.float32),
        grid=(R // blk,),
        in_specs=[
            pl.BlockSpec((1, blk, D), lambda i: (0, i, 0)),
            pl.BlockSpec((1, blk, D), lambda i: (1, i, 0)),
        ],
        out_specs=pl.BlockSpec((blk, D), lambda i: (i, 0)),
    )(partials, partials)
    return out


# speedup vs baseline: 1.1212x; 1.1212x over previous
import functools

import jax
import jax.numpy as jnp
from jax import lax
from jax.experimental import pallas as pl
from jax.experimental.pallas import tpu as pltpu
from jax.experimental.pallas import tpu_sc as plsc

E = 320000
R = 10000
RP = 10240
D = 128
NC = 2
NS = 16
NWORK = NC * NS
EPT = E // NWORK
W = 96
NB = 2
NFULL = EPT // W
REM = EPT - NFULL * W
NWIN = NFULL + (1 if REM else 0)
NWINP = -(-NWIN // NB) * NB
TAIL = EPT - W
RPT = RP // NS
LANES = 16
ZR = 16
NGRP = NWINP // NB


def _win_off(j):
    off = jnp.where(j < NFULL, j * W, TAIL if REM else 0)
    if REM:
        off = jnp.where(j > NFULL, 0, off)
    return off


def _sc_scatter_body(
    i1_hbm, idx_hbm, out_hbm, idx_v, upd_v, zrow_v, acc_sh, isem, gsems
):
    c = lax.axis_index("c")
    s = lax.axis_index("s")
    wid = c * NS + s
    ebase = wid * EPT

    idx_cp = pltpu.async_copy(idx_hbm.at[wid], idx_v, isem)
    prime = [
        pltpu.async_copy(
            i1_hbm.at[pl.ds(ebase + b * W, W)], upd_v.at[b], gsems[b]
        )
        for b in range(NB)
    ]

    def zero_row(i, carry):
        for blk in range(D // LANES):
            zrow_v[i, pl.ds(blk * LANES, LANES)] = jnp.zeros((LANES,), jnp.float32)
        return carry

    lax.fori_loop(0, ZR, zero_row, 0)
    for r in range(RPT // ZR):
        pltpu.sync_copy(zrow_v, acc_sh.at[pl.ds(s * RPT + r * ZR, ZR)])
    idx_cp.wait()
    plsc.subcore_barrier()

    def group(g, carry):
        for b in range(NB):
            j = g * NB + b
            prime[b].wait()
            pltpu.sync_copy(upd_v.at[b], acc_sh.at[idx_v.at[j]], add=True)
            off = _win_off(j + NB)
            pltpu.async_copy(
                i1_hbm.at[pl.ds(ebase + off, W)], upd_v.at[b], gsems[b]
            )
        return carry

    lax.fori_loop(0, NGRP - 1, group, 0)
    base = (NGRP - 1) * NB
    for b in range(NB):
        prime[b].wait()
        pltpu.sync_copy(upd_v.at[b], acc_sh.at[idx_v.at[base + b]], add=True)
    plsc.subcore_barrier()

    rbase = s * RPT
    pltpu.sync_copy(
        acc_sh.at[pl.ds(rbase, RPT)],
        out_hbm.at[c, pl.ds(rbase, RPT)],
    )


_sc_scatter = functools.partial(
    pl.kernel,
    out_type=jax.ShapeDtypeStruct((NC, RP, D), jnp.float32),
    mesh=plsc.VectorSubcoreMesh(
        core_axis_name="c", subcore_axis_name="s", num_cores=NC, num_subcores=NS
    ),
    scratch_types=[
        pltpu.VMEM((NWINP, W), jnp.int32),
        pltpu.VMEM((NB, W, D), jnp.float32),
        pltpu.VMEM((ZR, D), jnp.float32),
        pltpu.VMEM_SHARED((RP, D), jnp.float32),
        pltpu.SemaphoreType.DMA,
        [pltpu.SemaphoreType.DMA] * NB,
    ],
)(_sc_scatter_body)


def _sum_partials_body(a_ref, b_ref, o_ref):
    o_ref[...] = a_ref[0] + b_ref[0]


def kernel(i1, pair_i, p1):
    del p1
    pi = pair_i.astype(jnp.int32).reshape(NWORK, EPT)
    ntrash = (W - REM if REM else 0) + (NWINP - NWIN) * W
    parts = [pi[:, : NFULL * W].reshape(NWORK, NFULL, W)]
    if ntrash:
        trash = R + (
            jnp.arange(NWORK * ntrash, dtype=jnp.int32) % (RP - R)
        ).reshape(NWORK, ntrash)
        rest = jnp.concatenate([trash[:, : W - REM], pi[:, NFULL * W :]], axis=1) \
            if REM else trash[:, :0]
        rest = jnp.concatenate(
            [rest, trash[:, W - REM if REM else 0 :]], axis=1
        ).reshape(NWORK, NWINP - NFULL, W)
        parts.append(rest)
    idx = jnp.concatenate(parts, axis=1) if len(parts) > 1 else parts[0]
    partials = _sc_scatter(i1, idx)
    blk = 1000
    out = pl.pallas_call(
        _sum_partials_body,
        out_shape=jax.ShapeDtypeStruct((R, D), jnp.float32),
        grid=(R // blk,),
        in_specs=[
            pl.BlockSpec((1, blk, D), lambda i: (0, i, 0)),
            pl.BlockSpec((1, blk, D), lambda i: (1, i, 0)),
        ],
        out_specs=pl.BlockSpec((blk, D), lambda i: (i, 0)),
    )(partials, partials)
    return out


# speedup vs baseline: 1.1633x; 1.0375x over previous
import functools

import jax
import jax.numpy as jnp
from jax import lax
from jax.experimental import pallas as pl
from jax.experimental.pallas import tpu as pltpu
from jax.experimental.pallas import tpu_sc as plsc

E = 320000
R = 10000
RP = 10240
D = 128
NC = 2
NS = 16
NWORK = NC * NS
EPT = E // NWORK
W = 112
NB = 2
NFULL = EPT // W
REM = EPT - NFULL * W
NWIN = NFULL + (1 if REM else 0)
NWINP = -(-NWIN // NB) * NB
TAIL = EPT - W
RPT = RP // NS
LANES = 16
ZR = 16
NGRP = NWINP // NB


def _win_off(j):
    off = jnp.where(j < NFULL, j * W, TAIL if REM else 0)
    if REM:
        off = jnp.where(j > NFULL, 0, off)
    return off


def _sc_scatter_body(
    i1_hbm, idx_hbm, out_hbm, idx_v, upd_v, zrow_v, acc_sh, isem, gsems
):
    c = lax.axis_index("c")
    s = lax.axis_index("s")
    wid = c * NS + s
    ebase = wid * EPT

    idx_cp = pltpu.async_copy(idx_hbm.at[wid], idx_v, isem)
    prime = [
        pltpu.async_copy(
            i1_hbm.at[pl.ds(ebase + b * W, W)], upd_v.at[b], gsems[b]
        )
        for b in range(NB)
    ]

    def zero_row(i, carry):
        for blk in range(D // LANES):
            zrow_v[i, pl.ds(blk * LANES, LANES)] = jnp.zeros((LANES,), jnp.float32)
        return carry

    lax.fori_loop(0, ZR, zero_row, 0)
    for r in range(RPT // ZR):
        pltpu.sync_copy(zrow_v, acc_sh.at[pl.ds(s * RPT + r * ZR, ZR)])
    idx_cp.wait()
    plsc.subcore_barrier()

    def group(g, carry):
        for b in range(NB):
            j = g * NB + b
            prime[b].wait()
            pltpu.sync_copy(upd_v.at[b], acc_sh.at[idx_v.at[j]], add=True)
            off = _win_off(j + NB)
            pltpu.async_copy(
                i1_hbm.at[pl.ds(ebase + off, W)], upd_v.at[b], gsems[b]
            )
        return carry

    lax.fori_loop(0, NGRP - 1, group, 0)
    base = (NGRP - 1) * NB
    for b in range(NB):
        prime[b].wait()
        pltpu.sync_copy(upd_v.at[b], acc_sh.at[idx_v.at[base + b]], add=True)
    plsc.subcore_barrier()

    rbase = s * RPT
    pltpu.sync_copy(
        acc_sh.at[pl.ds(rbase, RPT)],
        out_hbm.at[c, pl.ds(rbase, RPT)],
    )


_sc_scatter = functools.partial(
    pl.kernel,
    out_type=jax.ShapeDtypeStruct((NC, RP, D), jnp.float32),
    mesh=plsc.VectorSubcoreMesh(
        core_axis_name="c", subcore_axis_name="s", num_cores=NC, num_subcores=NS
    ),
    scratch_types=[
        pltpu.VMEM((NWINP, W), jnp.int32),
        pltpu.VMEM((NB, W, D), jnp.float32),
        pltpu.VMEM((ZR, D), jnp.float32),
        pltpu.VMEM_SHARED((RP, D), jnp.float32),
        pltpu.SemaphoreType.DMA,
        [pltpu.SemaphoreType.DMA] * NB,
    ],
)(_sc_scatter_body)


def _sum_partials_body(a_ref, b_ref, o_ref):
    o_ref[...] = a_ref[0] + b_ref[0]


def kernel(i1, pair_i, p1):
    del p1
    pi = pair_i.astype(jnp.int32).reshape(NWORK, EPT)
    ntrash = (W - REM if REM else 0) + (NWINP - NWIN) * W
    parts = [pi[:, : NFULL * W].reshape(NWORK, NFULL, W)]
    if ntrash:
        trash = R + (
            jnp.arange(NWORK * ntrash, dtype=jnp.int32) % (RP - R)
        ).reshape(NWORK, ntrash)
        rest = jnp.concatenate([trash[:, : W - REM], pi[:, NFULL * W :]], axis=1) \
            if REM else trash[:, :0]
        rest = jnp.concatenate(
            [rest, trash[:, W - REM if REM else 0 :]], axis=1
        ).reshape(NWORK, NWINP - NFULL, W)
        parts.append(rest)
    idx = jnp.concatenate(parts, axis=1) if len(parts) > 1 else parts[0]
    partials = _sc_scatter(i1, idx)
    blk = 1000
    out = pl.pallas_call(
        _sum_partials_body,
        out_shape=jax.ShapeDtypeStruct((R, D), jnp.float32),
        grid=(R // blk,),
        in_specs=[
            pl.BlockSpec((1, blk, D), lambda i: (0, i, 0)),
            pl.BlockSpec((1, blk, D), lambda i: (1, i, 0)),
        ],
        out_specs=pl.BlockSpec((blk, D), lambda i: (i, 0)),
    )(partials, partials)
    return out


# speedup vs baseline: 1.1748x; 1.0100x over previous
import functools

import jax
import jax.numpy as jnp
from jax import lax
from jax.experimental import pallas as pl
from jax.experimental.pallas import tpu as pltpu
from jax.experimental.pallas import tpu_sc as plsc

E = 320000
R = 10000
RP = 10240
D = 128
NC = 2
NS = 16
NWORK = NC * NS
EPT = E // NWORK
W = 128
NB = 2
NFULL = EPT // W
REM = EPT - NFULL * W
NWIN = NFULL + (1 if REM else 0)
NWINP = -(-NWIN // NB) * NB
TAIL = EPT - W
RPT = RP // NS
LANES = 16
ZR = 16
NGRP = NWINP // NB


def _win_off(j):
    off = jnp.where(j < NFULL, j * W, TAIL if REM else 0)
    if REM:
        off = jnp.where(j > NFULL, 0, off)
    return off


def _sc_scatter_body(
    i1_hbm, idx_hbm, out_hbm, idx_v, upd_v, zrow_v, acc_sh, isem, gsems
):
    c = lax.axis_index("c")
    s = lax.axis_index("s")
    wid = c * NS + s
    ebase = wid * EPT

    idx_cp = pltpu.async_copy(idx_hbm.at[wid], idx_v, isem)
    prime = [
        pltpu.async_copy(
            i1_hbm.at[pl.ds(ebase + b * W, W)], upd_v.at[b], gsems[b]
        )
        for b in range(NB)
    ]

    def zero_row(i, carry):
        for blk in range(D // LANES):
            zrow_v[i, pl.ds(blk * LANES, LANES)] = jnp.zeros((LANES,), jnp.float32)
        return carry

    lax.fori_loop(0, ZR, zero_row, 0)
    for r in range(RPT // ZR):
        pltpu.sync_copy(zrow_v, acc_sh.at[pl.ds(s * RPT + r * ZR, ZR)])
    idx_cp.wait()
    plsc.subcore_barrier()

    def group(g, carry):
        for b in range(NB):
            j = g * NB + b
            prime[b].wait()
            pltpu.sync_copy(upd_v.at[b], acc_sh.at[idx_v.at[j]], add=True)
            off = _win_off(j + NB)
            pltpu.async_copy(
                i1_hbm.at[pl.ds(ebase + off, W)], upd_v.at[b], gsems[b]
            )
        return carry

    lax.fori_loop(0, NGRP - 1, group, 0)
    base = (NGRP - 1) * NB
    for b in range(NB):
        prime[b].wait()
        pltpu.sync_copy(upd_v.at[b], acc_sh.at[idx_v.at[base + b]], add=True)
    plsc.subcore_barrier()

    rbase = s * RPT
    pltpu.sync_copy(
        acc_sh.at[pl.ds(rbase, RPT)],
        out_hbm.at[c, pl.ds(rbase, RPT)],
    )


_sc_scatter = functools.partial(
    pl.kernel,
    out_type=jax.ShapeDtypeStruct((NC, RP, D), jnp.float32),
    mesh=plsc.VectorSubcoreMesh(
        core_axis_name="c", subcore_axis_name="s", num_cores=NC, num_subcores=NS
    ),
    scratch_types=[
        pltpu.VMEM((NWINP, W), jnp.int32),
        pltpu.VMEM((NB, W, D), jnp.float32),
        pltpu.VMEM((ZR, D), jnp.float32),
        pltpu.VMEM_SHARED((RP, D), jnp.float32),
        pltpu.SemaphoreType.DMA,
        [pltpu.SemaphoreType.DMA] * NB,
    ],
)(_sc_scatter_body)


def _sum_partials_body(a_ref, b_ref, o_ref):
    o_ref[...] = a_ref[0] + b_ref[0]


def kernel(i1, pair_i, p1):
    del p1
    pi = pair_i.astype(jnp.int32).reshape(NWORK, EPT)
    ntrash = (W - REM if REM else 0) + (NWINP - NWIN) * W
    parts = [pi[:, : NFULL * W].reshape(NWORK, NFULL, W)]
    if ntrash:
        trash = R + (
            jnp.arange(NWORK * ntrash, dtype=jnp.int32) % (RP - R)
        ).reshape(NWORK, ntrash)
        rest = jnp.concatenate([trash[:, : W - REM], pi[:, NFULL * W :]], axis=1) \
            if REM else trash[:, :0]
        rest = jnp.concatenate(
            [rest, trash[:, W - REM if REM else 0 :]], axis=1
        ).reshape(NWORK, NWINP - NFULL, W)
        parts.append(rest)
    idx = jnp.concatenate(parts, axis=1) if len(parts) > 1 else parts[0]
    partials = _sc_scatter(i1, idx)
    blk = 1000
    out = pl.pallas_call(
        _sum_partials_body,
        out_shape=jax.ShapeDtypeStruct((R, D), jnp.float32),
        grid=(R // blk,),
        in_specs=[
            pl.BlockSpec((1, blk, D), lambda i: (0, i, 0)),
            pl.BlockSpec((1, blk, D), lambda i: (1, i, 0)),
        ],
        out_specs=pl.BlockSpec((blk, D), lambda i: (i, 0)),
    )(partials, partials)
    return out


# speedup vs baseline: 1.1881x; 1.0113x over previous
import functools

import jax
import jax.numpy as jnp
from jax import lax
from jax.experimental import pallas as pl
from jax.experimental.pallas import tpu as pltpu
from jax.experimental.pallas import tpu_sc as plsc

E = 320000
R = 10000
RP = 10240
D = 128
NC = 2
NS = 16
NWORK = NC * NS
EPT = E // NWORK
W = 128
NB = 2
NFULL = EPT // W
REM = EPT - NFULL * W
NWIN = NFULL + (1 if REM else 0)
NWINP = -(-NWIN // NB) * NB
TAIL = EPT - W
RPT = RP // NS
LANES = 16
ZR = 16
NGRP = NWINP // NB


def _win_off(j):
    return jnp.where(j < NFULL, j * W, jnp.where(j == NFULL, TAIL, 0))


def _sc_scatter_body(
    i1_hbm, idx_hbm, out_hbm, idx_v, upd_v, zrow_v, acc_sh, isem, gsems
):
    c = lax.axis_index("c")
    s = lax.axis_index("s")
    wid = c * NS + s
    ebase = wid * EPT

    idx_cp = pltpu.async_copy(idx_hbm.at[wid], idx_v, isem)
    prime = [
        pltpu.async_copy(
            i1_hbm.at[pl.ds(ebase + b * W, W)], upd_v.at[b], gsems[b]
        )
        for b in range(NB)
    ]

    def zero_row(i, carry):
        for blk in range(D // LANES):
            zrow_v[i, pl.ds(blk * LANES, LANES)] = jnp.zeros((LANES,), jnp.float32)
        return carry

    lax.fori_loop(0, ZR, zero_row, 0)
    for r in range(RPT // ZR):
        pltpu.sync_copy(zrow_v, acc_sh.at[pl.ds(s * RPT + r * ZR, ZR)])
    idx_cp.wait()

    lane = lax.iota(jnp.int32, LANES)
    tail_vec = idx_v[NFULL, pl.ds(0, LANES)]
    idx_v[NFULL, pl.ds(W - REM, LANES)] = tail_vec
    for k in range((W - REM) // LANES):
        idx_v[NFULL, pl.ds(k * LANES, LANES)] = R + lane + k * LANES
    for k in range(W // LANES):
        idx_v[NFULL + 1, pl.ds(k * LANES, LANES)] = R + lane + (W - REM) + k * LANES
    plsc.subcore_barrier()

    def group(g, carry):
        for b in range(NB):
            j = g * NB + b
            prime[b].wait()
            pltpu.sync_copy(upd_v.at[b], acc_sh.at[idx_v.at[j]], add=True)
            off = _win_off(j + NB)
            pltpu.async_copy(
                i1_hbm.at[pl.ds(ebase + off, W)], upd_v.at[b], gsems[b]
            )
        return carry

    lax.fori_loop(0, NGRP - 1, group, 0)
    base = (NGRP - 1) * NB
    for b in range(NB):
        prime[b].wait()
        pltpu.sync_copy(upd_v.at[b], acc_sh.at[idx_v.at[base + b]], add=True)
    plsc.subcore_barrier()

    rbase = s * RPT
    pltpu.sync_copy(
        acc_sh.at[pl.ds(rbase, RPT)],
        out_hbm.at[c, pl.ds(rbase, RPT)],
    )


_sc_scatter = functools.partial(
    pl.kernel,
    out_type=jax.ShapeDtypeStruct((NC, RP, D), jnp.float32),
    mesh=plsc.VectorSubcoreMesh(
        core_axis_name="c", subcore_axis_name="s", num_cores=NC, num_subcores=NS
    ),
    scratch_types=[
        pltpu.VMEM((NWINP, W), jnp.int32),
        pltpu.VMEM((NB, W, D), jnp.float32),
        pltpu.VMEM((ZR, D), jnp.float32),
        pltpu.VMEM_SHARED((RP, D), jnp.float32),
        pltpu.SemaphoreType.DMA,
        [pltpu.SemaphoreType.DMA] * NB,
    ],
)(_sc_scatter_body)


def _sum_partials_body(a_ref, b_ref, o_ref):
    o_ref[...] = a_ref[0] + b_ref[0]


def kernel(i1, pair_i, p1):
    del p1
    idx = jnp.pad(
        pair_i.astype(jnp.int32).reshape(NWORK, EPT),
        ((0, 0), (0, NWINP * W - EPT)),
    ).reshape(NWORK, NWINP, W)
    partials = _sc_scatter(i1, idx)
    blk = 1000
    out = pl.pallas_call(
        _sum_partials_body,
        out_shape=jax.ShapeDtypeStruct((R, D), jnp.float32),
        grid=(R // blk,),
        in_specs=[
            pl.BlockSpec((1, blk, D), lambda i: (0, i, 0)),
            pl.BlockSpec((1, blk, D), lambda i: (1, i, 0)),
        ],
        out_specs=pl.BlockSpec((blk, D), lambda i: (i, 0)),
    )(partials, partials)
    return out


# speedup vs baseline: 1.2088x; 1.0174x over previous
import functools

import jax
import jax.numpy as jnp
from jax import lax
from jax.experimental import pallas as pl
from jax.experimental.pallas import tpu as pltpu
from jax.experimental.pallas import tpu_sc as plsc

E = 320000
R = 10000
RP = 10240
D = 128
NC = 2
NS = 16
NWORK = NC * NS
EPT = E // NWORK
W = 128
NB = 2
NFULL = EPT // W
REM = EPT - NFULL * W
NWIN = NFULL + (1 if REM else 0)
NWINP = -(-NWIN // NB) * NB
TAIL = EPT - W
RPT = RP // NS
LANES = 16
ZR = 16
NGRP = NWINP // NB


def _win_off(j):
    return jnp.where(j < NFULL, j * W, jnp.where(j == NFULL, TAIL, 0))


def _sc_scatter_body(
    i1_hbm, idx_hbm, out_hbm, idx_v, upd_v, acc_sh, isem, gsems
):
    c = lax.axis_index("c")
    s = lax.axis_index("s")
    wid = c * NS + s
    ebase = wid * EPT

    idx_cp = pltpu.async_copy(idx_hbm.at[wid], idx_v, isem)

    def zero_row(i, carry):
        for blk in range(D // LANES):
            upd_v[0, i, pl.ds(blk * LANES, LANES)] = jnp.zeros(
                (LANES,), jnp.float32
            )
        return carry

    lax.fori_loop(0, W, zero_row, 0)
    zcps = [
        pltpu.async_copy(
            upd_v.at[0], acc_sh.at[pl.ds(s * RPT + r * W, W)], gsems[0]
        )
        for r in range(RPT // W)
    ]
    for z in zcps:
        z.wait()

    prime = [
        pltpu.async_copy(
            i1_hbm.at[pl.ds(ebase + b * W, W)], upd_v.at[b], gsems[b]
        )
        for b in range(NB)
    ]
    idx_cp.wait()

    lane = lax.iota(jnp.int32, LANES)
    tail_vec = idx_v[NFULL, pl.ds(0, LANES)]
    idx_v[NFULL, pl.ds(W - REM, LANES)] = tail_vec
    for k in range((W - REM) // LANES):
        idx_v[NFULL, pl.ds(k * LANES, LANES)] = R + lane + k * LANES
    for k in range(W // LANES):
        idx_v[NFULL + 1, pl.ds(k * LANES, LANES)] = R + lane + (W - REM) + k * LANES
    plsc.subcore_barrier()

    def group(g, carry):
        for b in range(NB):
            j = g * NB + b
            prime[b].wait()
            pltpu.sync_copy(upd_v.at[b], acc_sh.at[idx_v.at[j]], add=True)
            off = _win_off(j + NB)
            pltpu.async_copy(
                i1_hbm.at[pl.ds(ebase + off, W)], upd_v.at[b], gsems[b]
            )
        return carry

    lax.fori_loop(0, NGRP - 1, group, 0)
    base = (NGRP - 1) * NB
    for b in range(NB):
        prime[b].wait()
        pltpu.sync_copy(upd_v.at[b], acc_sh.at[idx_v.at[base + b]], add=True)
    plsc.subcore_barrier()

    rbase = s * RPT
    pltpu.sync_copy(
        acc_sh.at[pl.ds(rbase, RPT)],
        out_hbm.at[c, pl.ds(rbase, RPT)],
    )


_sc_scatter = functools.partial(
    pl.kernel,
    out_type=jax.ShapeDtypeStruct((NC, RP, D), jnp.float32),
    mesh=plsc.VectorSubcoreMesh(
        core_axis_name="c", subcore_axis_name="s", num_cores=NC, num_subcores=NS
    ),
    scratch_types=[
        pltpu.VMEM((NWINP, W), jnp.int32),
        pltpu.VMEM((NB, W, D), jnp.float32),
        pltpu.VMEM_SHARED((RP, D), jnp.float32),
        pltpu.SemaphoreType.DMA,
        [pltpu.SemaphoreType.DMA] * NB,
    ],
)(_sc_scatter_body)


def _sum_partials_body(a_ref, b_ref, o_ref):
    o_ref[...] = a_ref[0] + b_ref[0]


def kernel(i1, pair_i, p1):
    del p1
    idx = jnp.pad(
        pair_i.astype(jnp.int32).reshape(NWORK, EPT),
        ((0, 0), (0, NWINP * W - EPT)),
    ).reshape(NWORK, NWINP, W)
    partials = _sc_scatter(i1, idx)
    blk = 2000
    out = pl.pallas_call(
        _sum_partials_body,
        out_shape=jax.ShapeDtypeStruct((R, D), jnp.float32),
        grid=(R // blk,),
        in_specs=[
            pl.BlockSpec((1, blk, D), lambda i: (0, i, 0)),
            pl.BlockSpec((1, blk, D), lambda i: (1, i, 0)),
        ],
        out_specs=pl.BlockSpec((blk, D), lambda i: (i, 0)),
    )(partials, partials)
    return out


# speedup vs baseline: 1.2244x; 1.0129x over previous
import functools

import jax
import jax.numpy as jnp
from jax import lax
from jax.experimental import pallas as pl
from jax.experimental.pallas import tpu as pltpu
from jax.experimental.pallas import tpu_sc as plsc

E = 320000
R = 10000
RP = 10240
D = 128
NC = 2
NS = 16
NWORK = NC * NS
EPT = E // NWORK
W = 128
NB = 2
NFULL = EPT // W
REM = EPT - NFULL * W
NWIN = NFULL + (1 if REM else 0)
NWINP = -(-NWIN // NB) * NB
TAIL = EPT - W
RPT = RP // NS
LANES = 16
ZR = 16
NGRP = NWINP // NB


def _win_off(j):
    return jnp.where(j < NFULL, j * W, jnp.where(j == NFULL, TAIL, 0))


def _sc_scatter_body(
    i1_hbm, idx_hbm, out_hbm, idx_v, upd_v, acc_sh, isem, gsems
):
    c = lax.axis_index("c")
    s = lax.axis_index("s")
    wid = c * NS + s
    ebase = wid * EPT

    idx_cp = pltpu.async_copy(idx_hbm.at[wid], idx_v, isem)

    def zero_row(i, carry):
        for blk in range(D // LANES):
            upd_v[0, i, pl.ds(blk * LANES, LANES)] = jnp.zeros(
                (LANES,), jnp.float32
            )
        return carry

    lax.fori_loop(0, W, zero_row, 0)
    zcps = [
        pltpu.async_copy(
            upd_v.at[0], acc_sh.at[pl.ds(s * RPT + r * W, W)], gsems[0]
        )
        for r in range(RPT // W)
    ]
    for z in zcps:
        z.wait()

    prime = [
        pltpu.async_copy(
            i1_hbm.at[pl.ds(ebase + b * W, W)], upd_v.at[b], gsems[b]
        )
        for b in range(NB)
    ]
    idx_cp.wait()

    lane = lax.iota(jnp.int32, LANES)
    tail_vec = idx_v[NFULL, pl.ds(0, LANES)]
    idx_v[NFULL, pl.ds(W - REM, LANES)] = tail_vec
    for k in range((W - REM) // LANES):
        idx_v[NFULL, pl.ds(k * LANES, LANES)] = R + lane + k * LANES
    for k in range(W // LANES):
        idx_v[NFULL + 1, pl.ds(k * LANES, LANES)] = R + lane + (W - REM) + k * LANES
    plsc.subcore_barrier()

    def group(g, carry):
        for b in range(NB):
            j = g * NB + b
            prime[b].wait()
            pltpu.sync_copy(upd_v.at[b], acc_sh.at[idx_v.at[j]], add=True)
            off = _win_off(j + NB)
            pltpu.async_copy(
                i1_hbm.at[pl.ds(ebase + off, W)], upd_v.at[b], gsems[b]
            )
        return carry

    lax.fori_loop(0, NGRP - 1, group, 0)
    base = (NGRP - 1) * NB
    for b in range(NB):
        prime[b].wait()
        pltpu.sync_copy(upd_v.at[b], acc_sh.at[idx_v.at[base + b]], add=True)
    plsc.subcore_barrier()

    rbase = s * RPT
    pltpu.sync_copy(
        acc_sh.at[pl.ds(rbase, RPT)],
        out_hbm.at[c, pl.ds(rbase, RPT)],
    )


_sc_scatter = functools.partial(
    pl.kernel,
    out_type=jax.ShapeDtypeStruct((NC, RP, D), jnp.float32),
    mesh=plsc.VectorSubcoreMesh(
        core_axis_name="c", subcore_axis_name="s", num_cores=NC, num_subcores=NS
    ),
    scratch_types=[
        pltpu.VMEM((NWINP, W), jnp.int32),
        pltpu.VMEM((NB, W, D), jnp.float32),
        pltpu.VMEM_SHARED((RP, D), jnp.float32),
        pltpu.SemaphoreType.DMA,
        [pltpu.SemaphoreType.DMA] * NB,
    ],
)(_sc_scatter_body)


def _sum_partials_body(a_ref, b_ref, o_ref):
    o_ref[...] = a_ref[0] + b_ref[0]


def kernel(i1, pair_i, p1):
    del p1
    idx = jnp.pad(
        pair_i.astype(jnp.int32).reshape(NWORK, EPT),
        ((0, 0), (0, NWINP * W - EPT)),
    ).reshape(NWORK, NWINP, W)
    partials = _sc_scatter(i1, idx)
    blk = 5000
    out = pl.pallas_call(
        _sum_partials_body,
        out_shape=jax.ShapeDtypeStruct((R, D), jnp.float32),
        grid=(R // blk,),
        in_specs=[
            pl.BlockSpec((1, blk, D), lambda i: (0, i, 0)),
            pl.BlockSpec((1, blk, D), lambda i: (1, i, 0)),
        ],
        out_specs=pl.BlockSpec((blk, D), lambda i: (i, 0)),
    )(partials, partials)
    return out


# speedup vs baseline: 1.2252x; 1.0007x over previous
import functools

import jax
import jax.numpy as jnp
from jax import lax
from jax.experimental import pallas as pl
from jax.experimental.pallas import tpu as pltpu
from jax.experimental.pallas import tpu_sc as plsc

E = 320000
R = 10000
RP = 10240
D = 128
NC = 2
NS = 16
NWORK = NC * NS
EPT = E // NWORK
W = 128
NB = 2
NFULL = EPT // W
REM = EPT - NFULL * W
NWIN = NFULL + (1 if REM else 0)
NWINP = -(-NWIN // NB) * NB
TAIL = EPT - W
RPT = RP // NS
LANES = 16
NGRP = NWINP // NB


def _win_off(j):
    return jnp.where(j < NFULL, j * W, jnp.where(j == NFULL, TAIL, 0))


def _sc_scatter_body(
    i1_hbm, idx_hbm, out_hbm, idx_v, upd_v, acc_sh, isem, gsems
):
    c = lax.axis_index("c")
    s = lax.axis_index("s")
    wid = c * NS + s
    ebase = wid * EPT

    idx_cp = pltpu.async_copy(idx_hbm.at[wid], idx_v, isem)

    def zero_row(i, carry):
        for blk in range(D // LANES):
            upd_v[0, i, pl.ds(blk * LANES, LANES)] = jnp.zeros(
                (LANES,), jnp.float32
            )
        return carry

    lax.fori_loop(0, W, zero_row, 0)
    zcps = [
        pltpu.async_copy(
            upd_v.at[0], acc_sh.at[pl.ds(s * RPT + r * W, W)], gsems[0]
        )
        for r in range(RPT // W)
    ]
    for z in zcps:
        z.wait()

    prime = [
        pltpu.async_copy(
            i1_hbm.at[pl.ds(ebase + b * W, W)], upd_v.at[b], gsems[b]
        )
        for b in range(NB)
    ]
    idx_cp.wait()

    lane = lax.iota(jnp.int32, LANES)
    tail_vec = idx_v[NFULL, pl.ds(0, LANES)]
    idx_v[NFULL, pl.ds(W - REM, LANES)] = tail_vec
    for k in range((W - REM) // LANES):
        idx_v[NFULL, pl.ds(k * LANES, LANES)] = R + lane + k * LANES
    for k in range(W // LANES):
        idx_v[NFULL + 1, pl.ds(k * LANES, LANES)] = R + lane + (W - REM) + k * LANES
    plsc.subcore_barrier()

    def group(g, carry):
        for b in range(NB):
            j = g * NB + b
            prime[b].wait()
            pltpu.sync_copy(upd_v.at[b], acc_sh.at[idx_v.at[j]], add=True)
            off = _win_off(j + NB)
            pltpu.async_copy(
                i1_hbm.at[pl.ds(ebase + off, W)], upd_v.at[b], gsems[b]
            )
        return carry

    lax.fori_loop(0, NGRP - 1, group, 0)
    base = (NGRP - 1) * NB
    for b in range(NB):
        prime[b].wait()
        pltpu.sync_copy(upd_v.at[b], acc_sh.at[idx_v.at[base + b]], add=True)
    plsc.subcore_barrier()

    rbase = s * RPT
    pltpu.sync_copy(
        acc_sh.at[pl.ds(rbase, RPT)],
        out_hbm.at[c, pl.ds(rbase, RPT)],
    )


_sc_scatter = functools.partial(
    pl.kernel,
    out_type=jax.ShapeDtypeStruct((NC, RP, D), jnp.float32),
    mesh=plsc.VectorSubcoreMesh(
        core_axis_name="c", subcore_axis_name="s", num_cores=NC, num_subcores=NS
    ),
    scratch_types=[
        pltpu.VMEM((NWINP, W), jnp.int32),
        pltpu.VMEM((NB, W, D), jnp.float32),
        pltpu.VMEM_SHARED((RP, D), jnp.float32),
        pltpu.SemaphoreType.DMA,
        [pltpu.SemaphoreType.DMA] * NB,
    ],
)(_sc_scatter_body)


def _sum_partials_body(a_ref, b_ref, o_ref):
    o_ref[...] = a_ref[0] + b_ref[0]


def kernel(i1, pair_i, p1):
    del p1
    idx = jnp.pad(
        pair_i.astype(jnp.int32).reshape(NWORK, EPT),
        ((0, 0), (0, NWINP * W - EPT)),
    ).reshape(NWORK, NWINP, W)
    partials = _sc_scatter(i1, idx)
    blk = 5000
    out = pl.pallas_call(
        _sum_partials_body,
        out_shape=jax.ShapeDtypeStruct((R, D), jnp.float32),
        grid=(R // blk,),
        in_specs=[
            pl.BlockSpec((1, blk, D), lambda i: (0, i, 0)),
            pl.BlockSpec((1, blk, D), lambda i: (1, i, 0)),
        ],
        out_specs=pl.BlockSpec((blk, D), lambda i: (i, 0)),
    )(partials, partials)
    return out
